# baseline (speedup 1.0000x reference)
import functools

import jax
import jax.numpy as jnp
from jax import lax
from jax.experimental import pallas as pl
from jax.experimental.pallas import tpu as pltpu
from jax.experimental.pallas import tpu_sc as plsc

_N_ROWS = 1000000
_BATCH = 16384
_DIM = 64
_PAD = 128
_NUM_CORES = 2
_NUM_SUBCORES = 16
_NUM_WORKERS = _NUM_CORES * _NUM_SUBCORES
_B_PER_W = _BATCH // _NUM_WORKERS
_SPAN = 128
_NSPAN = _B_PER_W // _SPAN
_DEPTH = 8


def _make_gather():
  mesh = plsc.VectorSubcoreMesh(core_axis_name="c", subcore_axis_name="s")

  @functools.partial(
      pl.kernel,
      mesh=mesh,
      out_type=jax.ShapeDtypeStruct((_BATCH, _PAD), jnp.float32),
      scratch_types=[
          pltpu.VMEM((_B_PER_W + 16,), jnp.int32),
          pltpu.VMEM((_DEPTH, _DIM, _PAD), jnp.float32),
          pltpu.VMEM((2, _SPAN, _PAD), jnp.float32),
      ] + [pltpu.SemaphoreType.DMA] * (_DEPTH + 2),
      compiler_params=pltpu.CompilerParams(
          use_tc_tiling_on_sc=True, needs_layout_passes=False),
  )
  def gather_kernel(tablet_hbm, idx_hbm, outp_hbm,
                    idx_v, grp, outstage, *sems_all):
    sems = sems_all[:_DEPTH]
    so = sems_all[_DEPTH:]
    wid = lax.axis_index("s") * _NUM_CORES + lax.axis_index("c")
    base = wid * _B_PER_W

    pltpu.sync_copy(idx_hbm.at[pl.ds(base, _B_PER_W)], idx_v.at[pl.ds(0, _B_PER_W)])

    def fetch(i, b):
      r = idx_v[pl.ds(i, 16)][0]
      off = pl.multiple_of(lax.shift_right_logical(r, 7) * _PAD, _PAD)
      for a in range(_DIM // 8):
        pltpu.async_copy(
            tablet_hbm.at[pl.ds(8 * a, 8), pl.ds(off, _PAD)],
            grp.at[b, pl.ds(8 * a, 8)], sems[b])

    def drain(b):
      pltpu.make_async_copy(
          tablet_hbm.at[:, pl.ds(0, _PAD)], grp.at[b], sems[b]).wait()

    def extract(i, o, b):
      r = idx_v[pl.ds(i, 16)][0]
      row = jnp.bitwise_and(i, _SPAN - 1)
      lv = jnp.full((16,), jnp.bitwise_and(r, 127), jnp.int32)
      for q in range(_DIM // 16):
        cv = lax.iota(jnp.int32, 16) + q * 16
        vals = plsc.load_gather(grp.at[b], [cv, lv])
        outstage[o, row, pl.ds(q * 16, 16)] = vals

    for b in range(_DEPTH):
      fetch(b, b)

    ho = [None, None]
    for s in range(_NSPAN):
      o = s & 1
      if ho[o] is not None:
        ho[o].wait()

      def body(g, carry, s=s, o=o):
        i0 = s * _SPAN + g * _DEPTH
        for b in range(_DEPTH):
          i = i0 + b
          drain(b)
          extract(i, o, b)

          @pl.when(i + _DEPTH < _B_PER_W)
          def _():
            fetch(i + _DEPTH, b)

        return carry

      lax.fori_loop(0, _SPAN // _DEPTH, body, 0)
      ho[o] = pltpu.async_copy(
          outstage.at[o], outp_hbm.at[pl.ds(base + s * _SPAN, _SPAN)], so[o])
    ho[0].wait()
    ho[1].wait()

  return gather_kernel


_gather = _make_gather()


@jax.jit
def kernel(y_true, proxies_weight):
  padded = _gather(proxies_weight.T, y_true.astype(jnp.int32))
  return padded[:, :_DIM]


# speedup vs baseline: 1.0080x; 1.0080x over previous
import functools

import jax
import jax.numpy as jnp
from jax import lax
from jax.experimental import pallas as pl
from jax.experimental.pallas import tpu as pltpu
from jax.experimental.pallas import tpu_sc as plsc

_N_ROWS = 1000000
_BATCH = 16384
_DIM = 64
_PAD = 128
_NUM_CORES = 2
_NUM_SUBCORES = 16
_NUM_WORKERS = _NUM_CORES * _NUM_SUBCORES
_B_PER_W = _BATCH // _NUM_WORKERS
_SPAN = 128
_NSPAN = _B_PER_W // _SPAN
_DEPTH = 8


def _make_gather():
  mesh = plsc.VectorSubcoreMesh(core_axis_name="c", subcore_axis_name="s")

  @functools.partial(
      pl.kernel,
      mesh=mesh,
      out_type=jax.ShapeDtypeStruct((_BATCH, _PAD), jnp.float32),
      scratch_types=[
          pltpu.VMEM((_B_PER_W + 16,), jnp.int32),
          pltpu.VMEM((_DEPTH, _DIM, _PAD), jnp.float32),
          pltpu.VMEM((2, _SPAN, _PAD), jnp.float32),
      ] + [pltpu.SemaphoreType.DMA] * (_DEPTH + 2),
      compiler_params=pltpu.CompilerParams(
          use_tc_tiling_on_sc=True, needs_layout_passes=False),
  )
  def gather_kernel(tablet_hbm, idx_hbm, outp_hbm,
                    idx_v, grp, outstage, *sems_all):
    sems = sems_all[:_DEPTH]
    so = sems_all[_DEPTH:]
    wid = lax.axis_index("s") * _NUM_CORES + lax.axis_index("c")
    base = wid * _B_PER_W

    pltpu.sync_copy(idx_hbm.at[pl.ds(base, _B_PER_W)], idx_v.at[pl.ds(0, _B_PER_W)])

    def fetch(i, b):
      r = idx_v[pl.ds(i, 16)][0]
      off = pl.multiple_of(lax.shift_right_logical(r, 7) * _PAD, _PAD)
      pltpu.async_copy(
          tablet_hbm.at[:, pl.ds(off, _PAD)], grp.at[b], sems[b])

    def drain(b):
      pltpu.make_async_copy(
          tablet_hbm.at[:, pl.ds(0, _PAD)], grp.at[b], sems[b]).wait()

    def extract(i, o, b):
      r = idx_v[pl.ds(i, 16)][0]
      row = jnp.bitwise_and(i, _SPAN - 1)
      lv = jnp.full((16,), jnp.bitwise_and(r, 127), jnp.int32)
      for q in range(_DIM // 16):
        cv = lax.iota(jnp.int32, 16) + q * 16
        vals = plsc.load_gather(grp.at[b], [cv, lv])
        outstage[o, row, pl.ds(q * 16, 16)] = vals

    for b in range(_DEPTH):
      fetch(b, b)

    ho = [None, None]
    for s in range(_NSPAN):
      o = s & 1
      if ho[o] is not None:
        ho[o].wait()

      def body(g, carry, s=s, o=o):
        i0 = s * _SPAN + g * _DEPTH
        for b in range(_DEPTH):
          i = i0 + b
          drain(b)
          extract(i, o, b)

          @pl.when(i + _DEPTH < _B_PER_W)
          def _():
            fetch(i + _DEPTH, b)

        return carry

      lax.fori_loop(0, _SPAN // _DEPTH, body, 0)
      ho[o] = pltpu.async_copy(
          outstage.at[o], outp_hbm.at[pl.ds(base + s * _SPAN, _SPAN)], so[o])
    ho[0].wait()
    ho[1].wait()

  return gather_kernel


_gather = _make_gather()


@jax.jit
def kernel(y_true, proxies_weight):
  padded = _gather(proxies_weight.T, y_true.astype(jnp.int32))
  return padded[:, :_DIM]
